# baseline (speedup 1.0000x reference)
import functools

import jax
import jax.numpy as jnp
from jax import lax
from jax.experimental import pallas as pl
from jax.experimental.pallas import tpu as pltpu
from jax.experimental.pallas import tpu_sc as plsc

N_NODES = 10000
N_P = 10240
N_EDGES = 320000
NC, NS = 2, 16
NW = NC * NS
CH = 80
NCH = 125
EW = NCH * CH
RPS = N_P // NS

_MESH = plsc.VectorSubcoreMesh(
    core_axis_name="c", subcore_axis_name="s", num_cores=NC, num_subcores=NS)
_SC_PARAMS = pltpu.CompilerParams(use_tc_tiling_on_sc=False)


def _deg_body(dst_hbm, ones_hbm, out_hbm, dst_b, ones_v, zero_v, acc_sh, ssem):
  cid = lax.axis_index("c")
  sid = lax.axis_index("s")
  wid = sid * NC + cid
  for j in range(RPS // 16):
    zero_v[pl.ds(j * 16, 16)] = jnp.zeros((16,), jnp.float32)
  pltpu.sync_copy(zero_v, acc_sh.at[pl.ds(sid * RPS, RPS)])
  pltpu.sync_copy(ones_hbm, ones_v)
  pltpu.sync_copy(dst_hbm.at[pl.ds(wid * NCH, NCH)], dst_b)
  plsc.subcore_barrier()

  def fire(i, c):
    pltpu.async_copy(ones_v, acc_sh.at[dst_b.at[i]], ssem, add=True)
    return c

  def drain(i, c):
    pltpu.make_async_copy(ones_v, acc_sh.at[dst_b.at[0]], ssem).wait()
    return c

  lax.fori_loop(0, NCH, fire, 0)
  lax.fori_loop(0, NCH, drain, 0)
  plsc.subcore_barrier()
  pltpu.sync_copy(acc_sh.at[pl.ds(sid * RPS, RPS)],
                  out_hbm.at[cid, pl.ds(sid * RPS, RPS)])


_deg_kernel = functools.partial(
    pl.kernel,
    out_type=jax.ShapeDtypeStruct((NC, N_P), jnp.float32),
    mesh=_MESH,
    compiler_params=_SC_PARAMS,
    scratch_types=[
        pltpu.VMEM((NCH, CH), jnp.int32),
        pltpu.VMEM((CH,), jnp.float32),
        pltpu.VMEM((RPS,), jnp.float32),
        pltpu.VMEM_SHARED((N_P,), jnp.float32),
        pltpu.SemaphoreType.DMA,
    ],
)(_deg_body)


def _make_agg(widths, depth, gahead, ch=CH, nch=NCH):
  nt = len(widths)

  IR = 6

  def body(*refs):
    src_hbm, dst_hbm = refs[0], refs[1]
    tables = refs[2:2 + nt]
    outs = refs[2 + nt:2 + 2 * nt]
    si = 2 + 2 * nt
    src_b, dst_b = refs[si], refs[si + 1]
    rows = refs[si + 2:si + 2 + nt]
    accs = refs[si + 2 + nt:si + 2 + 2 * nt]
    isem = refs[si + 2 + 2 * nt]
    gsem = refs[si + 3 + 2 * nt]
    ssem = refs[si + 4 + 2 * nt]

    cid = lax.axis_index("c")
    sid = lax.axis_index("s")
    wid = sid * NC + cid
    for t in range(nt):
      pltpu.sync_copy(tables[t].at[pl.ds(sid * RPS, RPS)],
                      accs[t].at[pl.ds(sid * RPS, RPS)])

    def ifire(i):
      slot = lax.rem(i, IR)
      pltpu.async_copy(src_hbm.at[wid * nch + i], src_b.at[slot],
                       isem.at[slot])
      pltpu.async_copy(dst_hbm.at[wid * nch + i], dst_b.at[slot],
                       isem.at[slot])

    def iwait(i):
      slot = lax.rem(i, IR)
      pltpu.make_async_copy(src_hbm.at[0], src_b.at[0], isem.at[slot]).wait()
      pltpu.make_async_copy(dst_hbm.at[0], dst_b.at[0], isem.at[slot]).wait()

    def gfire(i, slot):
      for t in range(nt):
        pltpu.async_copy(tables[t].at[src_b.at[lax.rem(i, IR)]],
                         rows[t].at[slot], gsem)

    def gwait():
      for t in range(nt):
        pltpu.make_async_copy(tables[t].at[src_b.at[0]], rows[t].at[0],
                              gsem).wait()

    def sfire(i, slot):
      for t in range(nt):
        pltpu.async_copy(rows[t].at[slot], accs[t].at[dst_b.at[lax.rem(i, IR)]],
                         ssem, add=True)

    def swait():
      for t in range(nt):
        pltpu.make_async_copy(rows[t].at[0], accs[t].at[dst_b.at[0]],
                              ssem).wait()

    T = depth - gahead
    for j in range(min(IR - 2, nch)):
      ifire(j)
    plsc.subcore_barrier()
    for g in range(min(gahead, nch)):
      iwait(g)
      gfire(g, g)

    def step(i, c):
      gwait()

      @pl.when(i >= T)
      def _():
        swait()
      sfire(i, lax.rem(i, depth))

      @pl.when(i + gahead < nch)
      def _():
        iwait(i + gahead)
        gfire(i + gahead, lax.rem(i + gahead, depth))

      @pl.when(i + IR - 2 < nch)
      def _():
        ifire(i + IR - 2)
      return c

    lax.fori_loop(0, nch, step, 0)
    for _ in range(T):
      swait()
    plsc.subcore_barrier()
    for t in range(nt):
      pltpu.sync_copy(accs[t].at[pl.ds(sid * RPS, RPS)],
                      outs[t].at[cid, pl.ds(sid * RPS, RPS)])

  return functools.partial(
      pl.kernel,
      out_type=[jax.ShapeDtypeStruct((NC, N_P, w), jnp.float32)
                for w in widths],
      mesh=_MESH,
      compiler_params=(pltpu.CompilerParams(use_tc_tiling_on_sc=True)
                       if all(w == 128 for w in widths) else _SC_PARAMS),
      scratch_types=(
          [pltpu.VMEM((IR, ch), jnp.int32)] * 2
          + [pltpu.VMEM((depth, ch, w), jnp.float32) for w in widths]
          + [pltpu.VMEM_SHARED((N_P, w), jnp.float32) for w in widths]
          + [pltpu.SemaphoreType.DMA((IR,)),
             pltpu.SemaphoreType.DMA, pltpu.SemaphoreType.DMA]
      ),
  )(body)


_agg1 = _make_agg([128, 16], 3, 2)
_agg2 = _make_agg([128], 4, 3)
_agg3 = _make_agg([64], 4, 3)


BS = 1024
NBLK = N_P // BS


def _prep_body(degp, feat, cond, wf, dinv_o, xs1_o, xc_o):
  deg = degp[0, :] + degp[1, :] + 1.0
  dv = jnp.broadcast_to(lax.rsqrt(deg)[:, None], (BS, 128))
  dinv_o[...] = dv
  xs1_o[...] = dv * jnp.dot(feat[...], wf[...],
                            preferred_element_type=jnp.float32)
  xc_o[...] = dv[:, :16] * cond[...]


def _mid_body(dinv, p1f, xs1, p1c, xc, wc, wh, bf, bc, xs2_o):
  dv = dinv[...]
  f2h = jnp.tanh(dv * (p1f[0] + p1f[1] - xs1[...]) + bf[...])
  cagg = dv[:, :16] * (p1c[0] + p1c[1] - xc[...])
  c2h = jnp.tanh(jnp.dot(cagg, wc[...], preferred_element_type=jnp.float32)
                 + bc[...])
  xw2 = (jnp.dot(f2h, wh[0], preferred_element_type=jnp.float32)
         + jnp.dot(c2h, wh[1], preferred_element_type=jnp.float32))
  xs2_o[...] = dv * xw2


def _lat_body(dinv, p2, xs2, wl, bh, xs3_o):
  dv = dinv[...]
  h = jnp.tanh(dv * (p2[0] + p2[1] - xs2[...]) + bh[...])
  xs3_o[...] = dv[:, :64] * jnp.dot(h, wl[...],
                                    preferred_element_type=jnp.float32)


def _fin_body(dinv, p3, xs3, bl, z_o):
  z_o[...] = dinv[:, :64] * (p3[0] + p3[1] - xs3[...]) + bl[...]


def _row_spec(w):
  return pl.BlockSpec((BS, w), lambda i: (i, 0))


def _part_spec(w):
  return pl.BlockSpec((NC, BS, w), lambda i: (0, i, 0))


def _full_spec(shape):
  return pl.BlockSpec(shape, lambda i: tuple(0 for _ in shape))


@jax.jit
def kernel(feature, condition, edge_index, W_f2h, b_f2h, W_c2h, b_c2h,
           W_h2h, b_h2h, W_h2l, b_h2l):
  f32 = jnp.float32
  src_p = edge_index[0].astype(jnp.int32).reshape(NW * NCH, CH)
  dst_p = edge_index[1].astype(jnp.int32).reshape(NW * NCH, CH)
  ones = jnp.ones((CH,), f32)
  bf = b_f2h.reshape(1, -1)
  bc = b_c2h.reshape(1, -1)
  bh = b_h2h.reshape(1, -1)
  bl = b_h2l.reshape(1, -1)
  wh2 = W_h2h.reshape(2, 128, 128)

  degp = _deg_kernel(dst_p, ones)

  dinv, xs1, xc = pl.pallas_call(
      _prep_body,
      grid=(NBLK,),
      in_specs=[pl.BlockSpec((NC, BS), lambda i: (0, i)),
                _row_spec(128), _row_spec(16), _full_spec((128, 128))],
      out_specs=[_row_spec(128), _row_spec(128), _row_spec(16)],
      out_shape=[jax.ShapeDtypeStruct((N_P, 128), f32),
                 jax.ShapeDtypeStruct((N_P, 128), f32),
                 jax.ShapeDtypeStruct((N_P, 16), f32)],
  )(degp, feature, condition, W_f2h)

  p1f, p1c = _agg1(src_p, dst_p, xs1, xc)

  xs2 = pl.pallas_call(
      _mid_body,
      grid=(NBLK,),
      in_specs=[_row_spec(128), _part_spec(128), _row_spec(128),
                _part_spec(16), _row_spec(16),
                _full_spec((16, 128)), _full_spec((2, 128, 128)),
                _full_spec((1, 128)), _full_spec((1, 128))],
      out_specs=_row_spec(128),
      out_shape=jax.ShapeDtypeStruct((N_P, 128), f32),
  )(dinv, p1f, xs1, p1c, xc, W_c2h, wh2, bf, bc)

  (p2,) = _agg2(src_p, dst_p, xs2)

  xs3 = pl.pallas_call(
      _lat_body,
      grid=(NBLK,),
      in_specs=[_row_spec(128), _part_spec(128), _row_spec(128),
                _full_spec((128, 64)), _full_spec((1, 128))],
      out_specs=_row_spec(64),
      out_shape=jax.ShapeDtypeStruct((N_P, 64), f32),
  )(dinv, p2, xs2, W_h2l, bh)

  (p3,) = _agg3(src_p, dst_p, xs3)

  z = pl.pallas_call(
      _fin_body,
      grid=(NBLK,),
      in_specs=[_row_spec(128), _part_spec(64), _row_spec(64),
                _full_spec((1, 64))],
      out_specs=_row_spec(64),
      out_shape=jax.ShapeDtypeStruct((N_P, 64), f32),
  )(dinv, p3, xs3, bl)

  return z[:N_NODES]


# speedup vs baseline: 1.0402x; 1.0402x over previous
import functools

import jax
import jax.numpy as jnp
from jax import lax
from jax.experimental import pallas as pl
from jax.experimental.pallas import tpu as pltpu
from jax.experimental.pallas import tpu_sc as plsc

N_NODES = 10000
N_P = 10240
N_EDGES = 320000
NC, NS = 2, 16
NW = NC * NS
CH = 120
NCH = 84
EW = NCH * CH
E_P = NW * EW
RPS = N_P // NS

_MESH = plsc.VectorSubcoreMesh(
    core_axis_name="c", subcore_axis_name="s", num_cores=NC, num_subcores=NS)
_SC_PARAMS = pltpu.CompilerParams(use_tc_tiling_on_sc=False)


def _deg_body(dst_hbm, ones_hbm, out_hbm, dst_b, ones_v, zero_v, acc_sh, ssem):
  cid = lax.axis_index("c")
  sid = lax.axis_index("s")
  wid = sid * NC + cid
  for j in range(RPS // 16):
    zero_v[pl.ds(j * 16, 16)] = jnp.zeros((16,), jnp.float32)
  pltpu.sync_copy(zero_v, acc_sh.at[pl.ds(sid * RPS, RPS)])
  pltpu.sync_copy(ones_hbm, ones_v)
  pltpu.sync_copy(dst_hbm.at[pl.ds(wid * NCH, NCH)], dst_b)
  plsc.subcore_barrier()

  def fire(i, c):
    pltpu.async_copy(ones_v, acc_sh.at[dst_b.at[i]], ssem, add=True)
    return c

  def drain(i, c):
    pltpu.make_async_copy(ones_v, acc_sh.at[dst_b.at[0]], ssem).wait()
    return c

  lax.fori_loop(0, NCH, fire, 0)
  lax.fori_loop(0, NCH, drain, 0)
  plsc.subcore_barrier()
  pltpu.sync_copy(acc_sh.at[pl.ds(sid * RPS, RPS)],
                  out_hbm.at[cid, pl.ds(sid * RPS, RPS)])


_deg_kernel = functools.partial(
    pl.kernel,
    out_type=jax.ShapeDtypeStruct((NC, N_P), jnp.float32),
    mesh=_MESH,
    compiler_params=_SC_PARAMS,
    scratch_types=[
        pltpu.VMEM((NCH, CH), jnp.int32),
        pltpu.VMEM((CH,), jnp.float32),
        pltpu.VMEM((RPS,), jnp.float32),
        pltpu.VMEM_SHARED((N_P,), jnp.float32),
        pltpu.SemaphoreType.DMA,
    ],
)(_deg_body)


def _make_agg(widths, depth, gahead, ch=CH, nch=NCH):
  nt = len(widths)

  IR = 6

  def body(*refs):
    src_hbm, dst_hbm = refs[0], refs[1]
    tables = refs[2:2 + nt]
    outs = refs[2 + nt:2 + 2 * nt]
    si = 2 + 2 * nt
    src_b, dst_b = refs[si], refs[si + 1]
    rows = refs[si + 2:si + 2 + nt]
    accs = refs[si + 2 + nt:si + 2 + 2 * nt]
    isem = refs[si + 2 + 2 * nt]
    gsem = refs[si + 3 + 2 * nt]
    ssem = refs[si + 4 + 2 * nt]

    cid = lax.axis_index("c")
    sid = lax.axis_index("s")
    wid = sid * NC + cid
    for t in range(nt):
      pltpu.sync_copy(tables[t].at[pl.ds(sid * RPS, RPS)],
                      accs[t].at[pl.ds(sid * RPS, RPS)])

    def ifire(i):
      slot = lax.rem(i, IR)
      pltpu.async_copy(src_hbm.at[wid * nch + i], src_b.at[slot],
                       isem.at[slot])
      pltpu.async_copy(dst_hbm.at[wid * nch + i], dst_b.at[slot],
                       isem.at[slot])

    def iwait(i):
      slot = lax.rem(i, IR)
      pltpu.make_async_copy(src_hbm.at[0], src_b.at[0], isem.at[slot]).wait()
      pltpu.make_async_copy(dst_hbm.at[0], dst_b.at[0], isem.at[slot]).wait()

    def gfire(i, slot):
      for t in range(nt):
        pltpu.async_copy(tables[t].at[src_b.at[lax.rem(i, IR)]],
                         rows[t].at[slot], gsem)

    def gwait():
      for t in range(nt):
        pltpu.make_async_copy(tables[t].at[src_b.at[0]], rows[t].at[0],
                              gsem).wait()

    def sfire(i, slot):
      for t in range(nt):
        pltpu.async_copy(rows[t].at[slot], accs[t].at[dst_b.at[lax.rem(i, IR)]],
                         ssem, add=True)

    def swait():
      for t in range(nt):
        pltpu.make_async_copy(rows[t].at[0], accs[t].at[dst_b.at[0]],
                              ssem).wait()

    T = depth - gahead
    for j in range(min(IR - 2, nch)):
      ifire(j)
    plsc.subcore_barrier()
    for g in range(min(gahead, nch)):
      iwait(g)
      gfire(g, g)

    def step(i, c):
      gwait()

      @pl.when(i >= T)
      def _():
        swait()
      sfire(i, lax.rem(i, depth))

      @pl.when(i + gahead < nch)
      def _():
        iwait(i + gahead)
        gfire(i + gahead, lax.rem(i + gahead, depth))

      @pl.when(i + IR - 2 < nch)
      def _():
        ifire(i + IR - 2)
      return c

    lax.fori_loop(0, nch, step, 0)
    for _ in range(T):
      swait()
    plsc.subcore_barrier()
    for t in range(nt):
      pltpu.sync_copy(accs[t].at[pl.ds(sid * RPS, RPS)],
                      outs[t].at[cid, pl.ds(sid * RPS, RPS)])

  return functools.partial(
      pl.kernel,
      out_type=[jax.ShapeDtypeStruct((NC, N_P, w), jnp.float32)
                for w in widths],
      mesh=_MESH,
      compiler_params=(pltpu.CompilerParams(use_tc_tiling_on_sc=True)
                       if all(w == 128 for w in widths) else _SC_PARAMS),
      scratch_types=(
          [pltpu.VMEM((IR, ch), jnp.int32)] * 2
          + [pltpu.VMEM((depth, ch, w), jnp.float32) for w in widths]
          + [pltpu.VMEM_SHARED((N_P, w), jnp.float32) for w in widths]
          + [pltpu.SemaphoreType.DMA((IR,)),
             pltpu.SemaphoreType.DMA, pltpu.SemaphoreType.DMA]
      ),
  )(body)


CH1, NCH1 = 80, 125
_agg1 = _make_agg([128, 16], 3, 2, CH1, NCH1)
_agg2 = _make_agg([128], 3, 2)
_agg3 = _make_agg([64], 4, 3)


BS = 1024
NBLK = N_P // BS


def _prep_body(degp, feat, cond, wf, dinv_o, xs1_o, xc_o):
  deg = degp[0, :] + degp[1, :] + 1.0
  dv = jnp.broadcast_to(lax.rsqrt(deg)[:, None], (BS, 128))
  dinv_o[...] = dv
  xs1_o[...] = dv * jnp.dot(feat[...], wf[...],
                            preferred_element_type=jnp.float32)
  xc_o[...] = dv[:, :16] * cond[...]


def _mid_body(dinv, p1f, xs1, p1c, xc, wc, wh, bf, bc, xs2_o):
  dv = dinv[...]
  f2h = jnp.tanh(dv * (p1f[0] + p1f[1] - xs1[...]) + bf[...])
  cagg = dv[:, :16] * (p1c[0] + p1c[1] - xc[...])
  c2h = jnp.tanh(jnp.dot(cagg, wc[...], preferred_element_type=jnp.float32)
                 + bc[...])
  xw2 = (jnp.dot(f2h, wh[0], preferred_element_type=jnp.float32)
         + jnp.dot(c2h, wh[1], preferred_element_type=jnp.float32))
  xs2_o[...] = dv * xw2


def _lat_body(dinv, p2, xs2, wl, bh, xs3_o):
  dv = dinv[...]
  h = jnp.tanh(dv * (p2[0] + p2[1] - xs2[...]) + bh[...])
  xs3_o[...] = dv[:, :64] * jnp.dot(h, wl[...],
                                    preferred_element_type=jnp.float32)


def _fin_body(dinv, p3, xs3, bl, z_o):
  z_o[...] = dinv[:, :64] * (p3[0] + p3[1] - xs3[...]) + bl[...]


def _row_spec(w):
  return pl.BlockSpec((BS, w), lambda i: (i, 0))


def _part_spec(w):
  return pl.BlockSpec((NC, BS, w), lambda i: (0, i, 0))


def _full_spec(shape):
  return pl.BlockSpec(shape, lambda i: tuple(0 for _ in shape))


@jax.jit
def kernel(feature, condition, edge_index, W_f2h, b_f2h, W_c2h, b_c2h,
           W_h2h, b_h2h, W_h2l, b_h2l):
  f32 = jnp.float32
  pad_idx = N_NODES + (jnp.arange(E_P - N_EDGES, dtype=jnp.int32)
                       % (N_P - N_NODES))
  src_p = jnp.concatenate([edge_index[0].astype(jnp.int32),
                           pad_idx]).reshape(NW * NCH, CH)
  dst_p = jnp.concatenate([edge_index[1].astype(jnp.int32),
                           pad_idx]).reshape(NW * NCH, CH)
  src1 = edge_index[0].astype(jnp.int32).reshape(NW * NCH1, CH1)
  dst1 = edge_index[1].astype(jnp.int32).reshape(NW * NCH1, CH1)
  ones = jnp.ones((CH,), f32)
  bf = b_f2h.reshape(1, -1)
  bc = b_c2h.reshape(1, -1)
  bh = b_h2h.reshape(1, -1)
  bl = b_h2l.reshape(1, -1)
  wh2 = W_h2h.reshape(2, 128, 128)

  degp = _deg_kernel(dst_p, ones)

  dinv, xs1, xc = pl.pallas_call(
      _prep_body,
      grid=(NBLK,),
      in_specs=[pl.BlockSpec((NC, BS), lambda i: (0, i)),
                _row_spec(128), _row_spec(16), _full_spec((128, 128))],
      out_specs=[_row_spec(128), _row_spec(128), _row_spec(16)],
      out_shape=[jax.ShapeDtypeStruct((N_P, 128), f32),
                 jax.ShapeDtypeStruct((N_P, 128), f32),
                 jax.ShapeDtypeStruct((N_P, 16), f32)],
  )(degp, feature, condition, W_f2h)

  p1f, p1c = _agg1(src1, dst1, xs1, xc)

  xs2 = pl.pallas_call(
      _mid_body,
      grid=(NBLK,),
      in_specs=[_row_spec(128), _part_spec(128), _row_spec(128),
                _part_spec(16), _row_spec(16),
                _full_spec((16, 128)), _full_spec((2, 128, 128)),
                _full_spec((1, 128)), _full_spec((1, 128))],
      out_specs=_row_spec(128),
      out_shape=jax.ShapeDtypeStruct((N_P, 128), f32),
  )(dinv, p1f, xs1, p1c, xc, W_c2h, wh2, bf, bc)

  (p2,) = _agg2(src_p, dst_p, xs2)

  xs3 = pl.pallas_call(
      _lat_body,
      grid=(NBLK,),
      in_specs=[_row_spec(128), _part_spec(128), _row_spec(128),
                _full_spec((128, 64)), _full_spec((1, 128))],
      out_specs=_row_spec(64),
      out_shape=jax.ShapeDtypeStruct((N_P, 64), f32),
  )(dinv, p2, xs2, W_h2l, bh)

  (p3,) = _agg3(src_p, dst_p, xs3)

  z = pl.pallas_call(
      _fin_body,
      grid=(NBLK,),
      in_specs=[_row_spec(128), _part_spec(64), _row_spec(64),
                _full_spec((1, 64))],
      out_specs=_row_spec(64),
      out_shape=jax.ShapeDtypeStruct((N_P, 64), f32),
  )(dinv, p3, xs3, bl)

  return z[:N_NODES]


# speedup vs baseline: 1.0621x; 1.0211x over previous
import functools

import jax
import jax.numpy as jnp
from jax import lax
from jax.experimental import pallas as pl
from jax.experimental.pallas import tpu as pltpu
from jax.experimental.pallas import tpu_sc as plsc

N_NODES = 10000
N_P = 10240
N_EDGES = 320000
NC, NS = 2, 16
NW = NC * NS
CH = 120
NCH = 84
EW = NCH * CH
E_P = NW * EW
RPS = N_P // NS

_MESH = plsc.VectorSubcoreMesh(
    core_axis_name="c", subcore_axis_name="s", num_cores=NC, num_subcores=NS)
_SC_PARAMS = pltpu.CompilerParams(use_tc_tiling_on_sc=False)


def _deg_body(dst_hbm, ones_hbm, out_hbm, dst_b, ones_v, zero_v, acc_sh, ssem):
  cid = lax.axis_index("c")
  sid = lax.axis_index("s")
  wid = sid * NC + cid
  for j in range(RPS // 16):
    zero_v[pl.ds(j * 16, 16)] = jnp.zeros((16,), jnp.float32)
  pltpu.sync_copy(zero_v, acc_sh.at[pl.ds(sid * RPS, RPS)])
  pltpu.sync_copy(ones_hbm, ones_v)
  pltpu.sync_copy(dst_hbm.at[pl.ds(wid * NCH, NCH)], dst_b)
  plsc.subcore_barrier()

  def fire(i, c):
    pltpu.async_copy(ones_v, acc_sh.at[dst_b.at[i]], ssem, add=True)
    return c

  def drain(i, c):
    pltpu.make_async_copy(ones_v, acc_sh.at[dst_b.at[0]], ssem).wait()
    return c

  lax.fori_loop(0, NCH, fire, 0)
  lax.fori_loop(0, NCH, drain, 0)
  plsc.subcore_barrier()
  pltpu.sync_copy(acc_sh.at[pl.ds(sid * RPS, RPS)],
                  out_hbm.at[cid, pl.ds(sid * RPS, RPS)])


_deg_kernel = functools.partial(
    pl.kernel,
    out_type=jax.ShapeDtypeStruct((NC, N_P), jnp.float32),
    mesh=_MESH,
    compiler_params=_SC_PARAMS,
    scratch_types=[
        pltpu.VMEM((NCH, CH), jnp.int32),
        pltpu.VMEM((CH,), jnp.float32),
        pltpu.VMEM((RPS,), jnp.float32),
        pltpu.VMEM_SHARED((N_P,), jnp.float32),
        pltpu.SemaphoreType.DMA,
    ],
)(_deg_body)


def _make_agg(widths, depth, gahead, ch=CH, nch=NCH):
  nt = len(widths)

  IR = 6

  def body(*refs):
    src_hbm, dst_hbm = refs[0], refs[1]
    tables = refs[2:2 + nt]
    outs = refs[2 + nt:2 + 2 * nt]
    si = 2 + 2 * nt
    src_b, dst_b = refs[si], refs[si + 1]
    rows = refs[si + 2:si + 2 + nt]
    accs = refs[si + 2 + nt:si + 2 + 2 * nt]
    isem = refs[si + 2 + 2 * nt]
    gsem = refs[si + 3 + 2 * nt]
    ssem = refs[si + 4 + 2 * nt]

    cid = lax.axis_index("c")
    sid = lax.axis_index("s")
    wid = sid * NC + cid
    for t in range(nt):
      pltpu.sync_copy(tables[t].at[pl.ds(sid * RPS, RPS)],
                      accs[t].at[pl.ds(sid * RPS, RPS)])

    def ifire(i):
      slot = lax.rem(i, IR)
      pltpu.async_copy(src_hbm.at[wid * nch + i], src_b.at[slot],
                       isem.at[slot])
      pltpu.async_copy(dst_hbm.at[wid * nch + i], dst_b.at[slot],
                       isem.at[slot])

    def iwait(i):
      slot = lax.rem(i, IR)
      pltpu.make_async_copy(src_hbm.at[0], src_b.at[0], isem.at[slot]).wait()
      pltpu.make_async_copy(dst_hbm.at[0], dst_b.at[0], isem.at[slot]).wait()

    def gfire(i, slot):
      for t in range(nt):
        pltpu.async_copy(tables[t].at[src_b.at[lax.rem(i, IR)]],
                         rows[t].at[slot], gsem)

    def gwait():
      for t in range(nt):
        pltpu.make_async_copy(tables[t].at[src_b.at[0]], rows[t].at[0],
                              gsem).wait()

    def sfire(i, slot):
      for t in range(nt):
        pltpu.async_copy(rows[t].at[slot], accs[t].at[dst_b.at[lax.rem(i, IR)]],
                         ssem, add=True)

    def swait():
      for t in range(nt):
        pltpu.make_async_copy(rows[t].at[0], accs[t].at[dst_b.at[0]],
                              ssem).wait()

    T = depth - gahead
    for j in range(min(IR - 2, nch)):
      ifire(j)
    plsc.subcore_barrier()
    for g in range(min(gahead, nch)):
      iwait(g)
      gfire(g, g)

    def step(i, c):
      gwait()

      @pl.when(i >= T)
      def _():
        swait()
      sfire(i, lax.rem(i, depth))

      @pl.when(i + gahead < nch)
      def _():
        iwait(i + gahead)
        gfire(i + gahead, lax.rem(i + gahead, depth))

      @pl.when(i + IR - 2 < nch)
      def _():
        ifire(i + IR - 2)
      return c

    lax.fori_loop(0, nch, step, 0)
    for _ in range(T):
      swait()
    plsc.subcore_barrier()
    for t in range(nt):
      pltpu.sync_copy(accs[t].at[pl.ds(sid * RPS, RPS)],
                      outs[t].at[cid, pl.ds(sid * RPS, RPS)])

  return functools.partial(
      pl.kernel,
      out_type=[jax.ShapeDtypeStruct((NC, N_P, w), jnp.float32)
                for w in widths],
      mesh=_MESH,
      compiler_params=(pltpu.CompilerParams(use_tc_tiling_on_sc=True)
                       if all(w == 128 for w in widths) else _SC_PARAMS),
      scratch_types=(
          [pltpu.VMEM((IR, ch), jnp.int32)] * 2
          + [pltpu.VMEM((depth, ch, w), jnp.float32) for w in widths]
          + [pltpu.VMEM_SHARED((N_P, w), jnp.float32) for w in widths]
          + [pltpu.SemaphoreType.DMA((IR,)),
             pltpu.SemaphoreType.DMA, pltpu.SemaphoreType.DMA]
      ),
  )(body)


CH1, NCH1 = 80, 125
_agg1 = _make_agg([128, 16], 3, 2, CH1, NCH1)
_agg2 = _make_agg([128], 3, 2)
_agg3 = _make_agg([64], 4, 3)


BS = 2048
NBLK = N_P // BS


def _prep_body(degp, feat, cond, wf, dinv_o, xs1_o, xc_o):
  deg = degp[0, :] + degp[1, :] + 1.0
  dv = jnp.broadcast_to(lax.rsqrt(deg)[:, None], (BS, 128))
  dinv_o[...] = dv
  xs1_o[...] = dv * jnp.dot(feat[...], wf[...],
                            preferred_element_type=jnp.float32)
  xc_o[...] = dv[:, :16] * cond[...]


def _mid_body(dinv, p1f, xs1, p1c, xc, wc, wh, bf, bc, xs2_o):
  dv = dinv[...]
  f2h = jnp.tanh(dv * (p1f[0] + p1f[1] - xs1[...]) + bf[...])
  cagg = dv[:, :16] * (p1c[0] + p1c[1] - xc[...])
  c2h = jnp.tanh(jnp.dot(cagg, wc[...], preferred_element_type=jnp.float32)
                 + bc[...])
  xw2 = (jnp.dot(f2h, wh[0], preferred_element_type=jnp.float32)
         + jnp.dot(c2h, wh[1], preferred_element_type=jnp.float32))
  xs2_o[...] = dv * xw2


def _lat_body(dinv, p2, xs2, wl, bh, xs3_o):
  dv = dinv[...]
  h = jnp.tanh(dv * (p2[0] + p2[1] - xs2[...]) + bh[...])
  xs3_o[...] = dv[:, :64] * jnp.dot(h, wl[...],
                                    preferred_element_type=jnp.float32)


def _fin_body(dinv, p3, xs3, bl, z_o):
  z_o[...] = dinv[:, :64] * (p3[0] + p3[1] - xs3[...]) + bl[...]


def _row_spec(w):
  return pl.BlockSpec((BS, w), lambda i: (i, 0))


def _part_spec(w):
  return pl.BlockSpec((NC, BS, w), lambda i: (0, i, 0))


def _full_spec(shape):
  return pl.BlockSpec(shape, lambda i: tuple(0 for _ in shape))


@jax.jit
def kernel(feature, condition, edge_index, W_f2h, b_f2h, W_c2h, b_c2h,
           W_h2h, b_h2h, W_h2l, b_h2l):
  f32 = jnp.float32
  pad_idx = N_NODES + (jnp.arange(E_P - N_EDGES, dtype=jnp.int32)
                       % (N_P - N_NODES))
  src_p = jnp.concatenate([edge_index[0].astype(jnp.int32),
                           pad_idx]).reshape(NW * NCH, CH)
  dst_p = jnp.concatenate([edge_index[1].astype(jnp.int32),
                           pad_idx]).reshape(NW * NCH, CH)
  src1 = edge_index[0].astype(jnp.int32).reshape(NW * NCH1, CH1)
  dst1 = edge_index[1].astype(jnp.int32).reshape(NW * NCH1, CH1)
  ones = jnp.ones((CH,), f32)
  bf = b_f2h.reshape(1, -1)
  bc = b_c2h.reshape(1, -1)
  bh = b_h2h.reshape(1, -1)
  bl = b_h2l.reshape(1, -1)
  wh2 = W_h2h.reshape(2, 128, 128)

  degp = _deg_kernel(dst_p, ones)

  dinv, xs1, xc = pl.pallas_call(
      _prep_body,
      grid=(NBLK,),
      in_specs=[pl.BlockSpec((NC, BS), lambda i: (0, i)),
                _row_spec(128), _row_spec(16), _full_spec((128, 128))],
      out_specs=[_row_spec(128), _row_spec(128), _row_spec(16)],
      out_shape=[jax.ShapeDtypeStruct((N_P, 128), f32),
                 jax.ShapeDtypeStruct((N_P, 128), f32),
                 jax.ShapeDtypeStruct((N_P, 16), f32)],
  )(degp, feature, condition, W_f2h)

  p1f, p1c = _agg1(src1, dst1, xs1, xc)

  xs2 = pl.pallas_call(
      _mid_body,
      grid=(NBLK,),
      in_specs=[_row_spec(128), _part_spec(128), _row_spec(128),
                _part_spec(16), _row_spec(16),
                _full_spec((16, 128)), _full_spec((2, 128, 128)),
                _full_spec((1, 128)), _full_spec((1, 128))],
      out_specs=_row_spec(128),
      out_shape=jax.ShapeDtypeStruct((N_P, 128), f32),
  )(dinv, p1f, xs1, p1c, xc, W_c2h, wh2, bf, bc)

  (p2,) = _agg2(src_p, dst_p, xs2)

  xs3 = pl.pallas_call(
      _lat_body,
      grid=(NBLK,),
      in_specs=[_row_spec(128), _part_spec(128), _row_spec(128),
                _full_spec((128, 64)), _full_spec((1, 128))],
      out_specs=_row_spec(64),
      out_shape=jax.ShapeDtypeStruct((N_P, 64), f32),
  )(dinv, p2, xs2, W_h2l, bh)

  (p3,) = _agg3(src_p, dst_p, xs3)

  z = pl.pallas_call(
      _fin_body,
      grid=(NBLK,),
      in_specs=[_row_spec(128), _part_spec(64), _row_spec(64),
                _full_spec((1, 64))],
      out_specs=_row_spec(64),
      out_shape=jax.ShapeDtypeStruct((N_P, 64), f32),
  )(dinv, p3, xs3, bl)

  return z[:N_NODES]


# speedup vs baseline: 1.0626x; 1.0005x over previous
import functools

import jax
import jax.numpy as jnp
from jax import lax
from jax.experimental import pallas as pl
from jax.experimental.pallas import tpu as pltpu
from jax.experimental.pallas import tpu_sc as plsc

N_NODES = 10000
N_P = 10240
N_EDGES = 320000
NC, NS = 2, 16
NW = NC * NS
CH = 120
NCH = 84
EW = NCH * CH
E_P = NW * EW
RPS = N_P // NS

_MESH = plsc.VectorSubcoreMesh(
    core_axis_name="c", subcore_axis_name="s", num_cores=NC, num_subcores=NS)
_SC_PARAMS = pltpu.CompilerParams(use_tc_tiling_on_sc=False)


def _deg_body(dst_hbm, ones_hbm, out_hbm, dst_b, ones_v, zero_v, acc_sh, ssem):
  cid = lax.axis_index("c")
  sid = lax.axis_index("s")
  wid = sid * NC + cid
  for j in range(RPS // 16):
    zero_v[pl.ds(j * 16, 16)] = jnp.zeros((16,), jnp.float32)
  pltpu.sync_copy(zero_v, acc_sh.at[pl.ds(sid * RPS, RPS)])
  pltpu.sync_copy(ones_hbm, ones_v)
  pltpu.sync_copy(dst_hbm.at[pl.ds(wid * NCH, NCH)], dst_b)
  plsc.subcore_barrier()

  def fire(i, c):
    pltpu.async_copy(ones_v, acc_sh.at[dst_b.at[i]], ssem, add=True)
    return c

  def drain(i, c):
    pltpu.make_async_copy(ones_v, acc_sh.at[dst_b.at[0]], ssem).wait()
    return c

  lax.fori_loop(0, NCH, fire, 0)
  lax.fori_loop(0, NCH, drain, 0)
  plsc.subcore_barrier()
  pltpu.sync_copy(acc_sh.at[pl.ds(sid * RPS, RPS)],
                  out_hbm.at[cid, pl.ds(sid * RPS, RPS)])


_deg_kernel = functools.partial(
    pl.kernel,
    out_type=jax.ShapeDtypeStruct((NC, N_P), jnp.float32),
    mesh=_MESH,
    compiler_params=_SC_PARAMS,
    scratch_types=[
        pltpu.VMEM((NCH, CH), jnp.int32),
        pltpu.VMEM((CH,), jnp.float32),
        pltpu.VMEM((RPS,), jnp.float32),
        pltpu.VMEM_SHARED((N_P,), jnp.float32),
        pltpu.SemaphoreType.DMA,
    ],
)(_deg_body)


def _make_agg(widths, depth, gahead, ch=CH, nch=NCH):
  nt = len(widths)

  IR = 6

  def body(*refs):
    src_hbm, dst_hbm = refs[0], refs[1]
    tables = refs[2:2 + nt]
    outs = refs[2 + nt:2 + 2 * nt]
    si = 2 + 2 * nt
    src_b, dst_b = refs[si], refs[si + 1]
    rows = refs[si + 2:si + 2 + nt]
    accs = refs[si + 2 + nt:si + 2 + 2 * nt]
    isem = refs[si + 2 + 2 * nt]
    gsem = refs[si + 3 + 2 * nt]
    ssem = refs[si + 4 + 2 * nt]

    cid = lax.axis_index("c")
    sid = lax.axis_index("s")
    wid = sid * NC + cid
    for t in range(nt):
      pltpu.sync_copy(tables[t].at[pl.ds(sid * RPS, RPS)],
                      accs[t].at[pl.ds(sid * RPS, RPS)])

    def ifire(i):
      slot = lax.rem(i, IR)
      pltpu.async_copy(src_hbm.at[wid * nch + i], src_b.at[slot],
                       isem.at[slot])
      pltpu.async_copy(dst_hbm.at[wid * nch + i], dst_b.at[slot],
                       isem.at[slot])

    def iwait(i):
      slot = lax.rem(i, IR)
      pltpu.make_async_copy(src_hbm.at[0], src_b.at[0], isem.at[slot]).wait()
      pltpu.make_async_copy(dst_hbm.at[0], dst_b.at[0], isem.at[slot]).wait()

    def gfire(i, slot):
      for t in range(nt):
        pltpu.async_copy(tables[t].at[src_b.at[lax.rem(i, IR)]],
                         rows[t].at[slot], gsem)

    def gwait():
      for t in range(nt):
        pltpu.make_async_copy(tables[t].at[src_b.at[0]], rows[t].at[0],
                              gsem).wait()

    def sfire(i, slot):
      for t in range(nt):
        pltpu.async_copy(rows[t].at[slot], accs[t].at[dst_b.at[lax.rem(i, IR)]],
                         ssem, add=True)

    def swait():
      for t in range(nt):
        pltpu.make_async_copy(rows[t].at[0], accs[t].at[dst_b.at[0]],
                              ssem).wait()

    T = depth - gahead
    for j in range(min(IR - 2, nch)):
      ifire(j)
    plsc.subcore_barrier()
    for g in range(min(gahead, nch)):
      iwait(g)
      gfire(g, g)

    def step(i, c):
      gwait()

      @pl.when(i >= T)
      def _():
        swait()
      sfire(i, lax.rem(i, depth))

      @pl.when(i + gahead < nch)
      def _():
        iwait(i + gahead)
        gfire(i + gahead, lax.rem(i + gahead, depth))

      @pl.when(i + IR - 2 < nch)
      def _():
        ifire(i + IR - 2)
      return c

    lax.fori_loop(0, nch, step, 0)
    for _ in range(T):
      swait()
    plsc.subcore_barrier()
    for t in range(nt):
      pltpu.sync_copy(accs[t].at[pl.ds(sid * RPS, RPS)],
                      outs[t].at[cid, pl.ds(sid * RPS, RPS)])

  return functools.partial(
      pl.kernel,
      out_type=[jax.ShapeDtypeStruct((NC, N_P, w), jnp.float32)
                for w in widths],
      mesh=_MESH,
      compiler_params=(pltpu.CompilerParams(use_tc_tiling_on_sc=True)
                       if all(w == 128 for w in widths) else _SC_PARAMS),
      scratch_types=(
          [pltpu.VMEM((IR, ch), jnp.int32)] * 2
          + [pltpu.VMEM((depth, ch, w), jnp.float32) for w in widths]
          + [pltpu.VMEM_SHARED((N_P, w), jnp.float32) for w in widths]
          + [pltpu.SemaphoreType.DMA((IR,)),
             pltpu.SemaphoreType.DMA, pltpu.SemaphoreType.DMA]
      ),
  )(body)


CH1, NCH1 = 80, 125
_agg1 = _make_agg([128, 16], 3, 2, CH1, NCH1)
_agg2 = _make_agg([128], 3, 2)
_agg3 = _make_agg([64], 4, 3)


BS = 2560
NBLK = N_P // BS


def _prep_body(degp, feat, cond, wf, dinv_o, xs1_o, xc_o):
  deg = degp[0, :] + degp[1, :] + 1.0
  dv = jnp.broadcast_to(lax.rsqrt(deg)[:, None], (BS, 128))
  dinv_o[...] = dv
  xs1_o[...] = dv * jnp.dot(feat[...], wf[...],
                            preferred_element_type=jnp.float32)
  xc_o[...] = dv[:, :16] * cond[...]


def _mid_body(dinv, p1f, xs1, p1c, xc, wc, wh, bf, bc, xs2_o):
  dv = dinv[...]
  f2h = jnp.tanh(dv * (p1f[0] + p1f[1] - xs1[...]) + bf[...])
  cagg = dv[:, :16] * (p1c[0] + p1c[1] - xc[...])
  c2h = jnp.tanh(jnp.dot(cagg, wc[...], preferred_element_type=jnp.float32)
                 + bc[...])
  xw2 = (jnp.dot(f2h, wh[0], preferred_element_type=jnp.float32)
         + jnp.dot(c2h, wh[1], preferred_element_type=jnp.float32))
  xs2_o[...] = dv * xw2


def _lat_body(dinv, p2, xs2, wl, bh, xs3_o):
  dv = dinv[...]
  h = jnp.tanh(dv * (p2[0] + p2[1] - xs2[...]) + bh[...])
  xs3_o[...] = dv[:, :64] * jnp.dot(h, wl[...],
                                    preferred_element_type=jnp.float32)


def _fin_body(dinv, p3, xs3, bl, z_o):
  z_o[...] = dinv[:, :64] * (p3[0] + p3[1] - xs3[...]) + bl[...]


def _row_spec(w):
  return pl.BlockSpec((BS, w), lambda i: (i, 0))


def _part_spec(w):
  return pl.BlockSpec((NC, BS, w), lambda i: (0, i, 0))


def _full_spec(shape):
  return pl.BlockSpec(shape, lambda i: tuple(0 for _ in shape))


@jax.jit
def kernel(feature, condition, edge_index, W_f2h, b_f2h, W_c2h, b_c2h,
           W_h2h, b_h2h, W_h2l, b_h2l):
  f32 = jnp.float32
  pad_idx = N_NODES + (jnp.arange(E_P - N_EDGES, dtype=jnp.int32)
                       % (N_P - N_NODES))
  src_p = jnp.concatenate([edge_index[0].astype(jnp.int32),
                           pad_idx]).reshape(NW * NCH, CH)
  dst_p = jnp.concatenate([edge_index[1].astype(jnp.int32),
                           pad_idx]).reshape(NW * NCH, CH)
  src1 = edge_index[0].astype(jnp.int32).reshape(NW * NCH1, CH1)
  dst1 = edge_index[1].astype(jnp.int32).reshape(NW * NCH1, CH1)
  ones = jnp.ones((CH,), f32)
  bf = b_f2h.reshape(1, -1)
  bc = b_c2h.reshape(1, -1)
  bh = b_h2h.reshape(1, -1)
  bl = b_h2l.reshape(1, -1)
  wh2 = W_h2h.reshape(2, 128, 128)

  degp = _deg_kernel(dst_p, ones)

  dinv, xs1, xc = pl.pallas_call(
      _prep_body,
      grid=(NBLK,),
      in_specs=[pl.BlockSpec((NC, BS), lambda i: (0, i)),
                _row_spec(128), _row_spec(16), _full_spec((128, 128))],
      out_specs=[_row_spec(128), _row_spec(128), _row_spec(16)],
      out_shape=[jax.ShapeDtypeStruct((N_P, 128), f32),
                 jax.ShapeDtypeStruct((N_P, 128), f32),
                 jax.ShapeDtypeStruct((N_P, 16), f32)],
  )(degp, feature, condition, W_f2h)

  p1f, p1c = _agg1(src1, dst1, xs1, xc)

  xs2 = pl.pallas_call(
      _mid_body,
      grid=(NBLK,),
      in_specs=[_row_spec(128), _part_spec(128), _row_spec(128),
                _part_spec(16), _row_spec(16),
                _full_spec((16, 128)), _full_spec((2, 128, 128)),
                _full_spec((1, 128)), _full_spec((1, 128))],
      out_specs=_row_spec(128),
      out_shape=jax.ShapeDtypeStruct((N_P, 128), f32),
  )(dinv, p1f, xs1, p1c, xc, W_c2h, wh2, bf, bc)

  (p2,) = _agg2(src_p, dst_p, xs2)

  xs3 = pl.pallas_call(
      _lat_body,
      grid=(NBLK,),
      in_specs=[_row_spec(128), _part_spec(128), _row_spec(128),
                _full_spec((128, 64)), _full_spec((1, 128))],
      out_specs=_row_spec(64),
      out_shape=jax.ShapeDtypeStruct((N_P, 64), f32),
  )(dinv, p2, xs2, W_h2l, bh)

  (p3,) = _agg3(src_p, dst_p, xs3)

  z = pl.pallas_call(
      _fin_body,
      grid=(NBLK,),
      in_specs=[_row_spec(128), _part_spec(64), _row_spec(64),
                _full_spec((1, 64))],
      out_specs=_row_spec(64),
      out_shape=jax.ShapeDtypeStruct((N_P, 64), f32),
  )(dinv, p3, xs3, bl)

  return z[:N_NODES]
